# baseline (speedup 1.0000x reference)
import functools

import jax
import jax.numpy as jnp
from jax import lax
from jax.experimental import pallas as pl
from jax.experimental.pallas import tpu as pltpu
from jax.experimental.pallas import tpu_sc as plsc

_CHUNK = 64
_CPB = 32
_BLK = _CHUNK * _CPB
_NTILES = 16
_F = 128


def _coalesce_weights(edge_index, num_nodes):
    src, dst = edge_index[0], edge_index[1]
    valid = src != dst
    kf = (src * num_nodes + dst) * 4 + jnp.where(valid, 2, 0).astype(jnp.int32)
    kr = (dst * num_nodes + src) * 4 + jnp.where(valid, 1, 0).astype(jnp.int32)
    sp = jnp.sort(jnp.concatenate([kf, kr]))
    m = sp.shape[0]
    key = sp >> 2
    tag = sp & 3
    sym_inc = (tag + 1) >> 1
    theta_inc = (tag >> 1) - (tag & 1)

    iota = jnp.arange(m, dtype=jnp.int32)
    is_last = jnp.concatenate([key[1:] != key[:-1], jnp.ones((1,), bool)])
    cs = jnp.cumsum(sym_inc, dtype=jnp.int32)
    ctp = jnp.cumsum(theta_inc + 1, dtype=jnp.int32)

    neg1 = jnp.full((1,), -1, jnp.int32)
    prev_cs = lax.cummax(jnp.concatenate([neg1, jnp.where(is_last, cs, -1)[:-1]]))
    prev_ct = lax.cummax(jnp.concatenate([neg1, jnp.where(is_last, ctp, -1)[:-1]]))
    prev_ix = lax.cummax(jnp.concatenate([neg1, jnp.where(is_last, iota, -1)[:-1]]))

    sym = cs - jnp.maximum(prev_cs, 0)
    theta = (ctp - jnp.maximum(prev_ct, 0)) - (iota - prev_ix)

    row = key // num_nodes
    col = key - row * num_nodes
    base = sym.astype(jnp.float32) * -0.5
    t4 = theta & 3
    cosv = jnp.where(t4 == 0, 1.0, jnp.where(t4 == 2, -1.0, 0.0))
    sinv = jnp.where(t4 == 1, 1.0, jnp.where(t4 == 3, -1.0, 0.0))
    wA = jnp.where(is_last, base * cosv, 0.0)
    wB = jnp.where(is_last, base * sinv, 0.0)
    return row, col, wA, wB


def _propagate_sc(xs2, dinv2, row_m, col_m, w_m, num_nodes):
    nrows = row_m.shape[0]
    nblk = nrows // (_CPB * _NTILES)
    npad = _NTILES * 640
    stripe = npad // _NTILES
    piece = _CHUNK
    mesh = plsc.VectorSubcoreMesh(core_axis_name="c", subcore_axis_name="s")

    @functools.partial(
        pl.kernel,
        mesh=mesh,
        compiler_params=pltpu.CompilerParams(needs_layout_passes=False),
        out_type=jax.ShapeDtypeStruct((2 * npad, _F), jnp.float32),
        scratch_types=[
            pltpu.VMEM_SHARED((npad, _F), jnp.float32),
            pltpu.VMEM((num_nodes,), jnp.float32),
            pltpu.VMEM((_CPB, _CHUNK), jnp.int32),
            pltpu.VMEM((_CPB, _CHUNK), jnp.int32),
            pltpu.VMEM((_CPB, _CHUNK), jnp.float32),
            pltpu.VMEM((_CHUNK,), jnp.float32),
            pltpu.VMEM((_CHUNK, _F), jnp.float32),
            pltpu.VMEM((_CHUNK, _F), jnp.float32),
            pltpu.SemaphoreType.DMA,
            pltpu.SemaphoreType.DMA,
            pltpu.SemaphoreType.DMA,
            pltpu.SemaphoreType.DMA,
        ],
    )
    def prop(xs_hbm, dinv_hbm, row_hbm, col_hbm, w_hbm, out_hbm,
             acc_v, dinv_v, rowb_v, colb_v, wb_v, wbuf_v, buf_a, buf_b,
             sga, sgb, ssa, ssb):
        c = lax.axis_index("c")
        s = lax.axis_index("s")
        pltpu.sync_copy(dinv_hbm, dinv_v)
        xsc_hbm = xs_hbm.at[c]

        def zrow(i, carry):
            for k in range(_F // 16):
                buf_a[i, pl.ds(k * 16, 16)] = jnp.zeros((16,), jnp.float32)
            return carry
        lax.fori_loop(0, piece, zrow, 0)
        for p in range(stripe // piece):
            pltpu.sync_copy(buf_a, acc_v.at[pl.ds(s * stripe + p * piece, piece)])
        plsc.subcore_barrier()

        def start_gather(j, buf, sem):
            pltpu.async_copy(xsc_hbm.at[rowb_v.at[j]], buf, sem)

        def wait_gather(j, buf, sem):
            pltpu.make_async_copy(xsc_hbm.at[rowb_v.at[j]], buf, sem).wait()

        def start_scatter(j, buf, sem):
            pltpu.async_copy(buf, acc_v.at[colb_v.at[j]], sem, add=True)

        def wait_scatter(j, buf, sem):
            pltpu.make_async_copy(buf, acc_v.at[colb_v.at[j]], sem).wait()

        def compute_w(j):
            for k in range(_CHUNK // 16):
                ri = rowb_v[j, pl.ds(k * 16, 16)]
                ci = colb_v[j, pl.ds(k * 16, 16)]
                dr = plsc.load_gather(dinv_v, [ri])
                dc = plsc.load_gather(dinv_v, [ci])
                wbuf_v[pl.ds(k * 16, 16)] = wb_v[j, pl.ds(k * 16, 16)] * dr * dc

        def scale(buf):
            def scale_group(g, carry3):
                w16 = wbuf_v[pl.ds(g * 16, 16)]
                for lane in range(16):
                    i = g * 16 + lane
                    wsc = w16[lane]
                    for k in range(_F // 16):
                        buf[i, pl.ds(k * 16, 16)] = buf[i, pl.ds(k * 16, 16)] * wsc
                return carry3
            lax.fori_loop(0, _CHUNK // 16, scale_group, 0)

        def block_body(b, carry):
            rb = (s * nblk + b) * _CPB
            pltpu.sync_copy(row_hbm.at[pl.ds(rb, _CPB)], rowb_v)
            pltpu.sync_copy(col_hbm.at[pl.ds(rb, _CPB)], colb_v)
            pltpu.sync_copy(w_hbm.at[c, pl.ds(rb, _CPB)], wb_v)

            start_gather(0, buf_a, sga)
            compute_w(0)
            wait_gather(0, buf_a, sga)
            scale(buf_a)
            start_scatter(0, buf_a, ssa)
            start_gather(1, buf_b, sgb)

            def steady(j2, carry2):
                j = 2 * j2 + 1
                wait_gather(j, buf_b, sgb)
                compute_w(j)
                wait_scatter(j, buf_a, ssa)
                start_gather(j + 1, buf_a, sga)
                scale(buf_b)
                start_scatter(j, buf_b, ssb)
                jj = j + 1
                wait_gather(jj, buf_a, sga)
                compute_w(jj)
                wait_scatter(jj, buf_b, ssb)
                start_gather(jj + 1, buf_b, sgb)
                scale(buf_a)
                start_scatter(jj, buf_a, ssa)
                return carry2
            lax.fori_loop(0, (_CPB - 2) // 2, steady, 0)

            jl = _CPB - 1
            wait_gather(jl, buf_b, sgb)
            compute_w(jl)
            wait_scatter(jl, buf_a, ssa)
            scale(buf_b)
            start_scatter(jl, buf_b, ssb)
            wait_scatter(jl, buf_b, ssb)
            return carry
        lax.fori_loop(0, nblk, block_body, 0)
        plsc.subcore_barrier()
        for p in range(stripe // piece):
            r0 = s * stripe + p * piece
            pltpu.sync_copy(acc_v.at[pl.ds(r0, piece)],
                            out_hbm.at[pl.ds(c * npad + r0, piece)])

    return prop(xs2, dinv2, row_m, col_m, w_m)


def _dense_body(xr_ref, xi_ref, pa_ref, pb_ref, w0_ref, w1_ref, w01_ref,
                bias_ref, or_ref, oi_ref):
    a = jnp.dot(xr_ref[...], w0_ref[...], preferred_element_type=jnp.float32)
    a += jnp.dot(pa_ref[...], w1_ref[...], preferred_element_type=jnp.float32)
    b = jnp.dot(xi_ref[...], w01_ref[...], preferred_element_type=jnp.float32)
    b += jnp.dot(pb_ref[...], w1_ref[...], preferred_element_type=jnp.float32)
    bias = bias_ref[...]
    or_ref[...] = a - b + bias
    oi_ref[...] = a + b + bias


def _dense_stage(x_real, x_imag, pa, pb, w0, w1, bias):
    n, f = x_real.shape
    blk = 2000
    grid = n // blk
    bspec_x = pl.BlockSpec((blk, f), lambda i: (i, 0))
    bspec_w = pl.BlockSpec((f, f), lambda i: (0, 0))
    bspec_b = pl.BlockSpec((1, f), lambda i: (0, 0))
    return pl.pallas_call(
        _dense_body,
        grid=(grid,),
        in_specs=[bspec_x, bspec_x, bspec_x, bspec_x, bspec_w, bspec_w,
                  bspec_w, bspec_b],
        out_specs=[bspec_x, bspec_x],
        out_shape=[jax.ShapeDtypeStruct((n, f), jnp.float32),
                   jax.ShapeDtypeStruct((n, f), jnp.float32)],
    )(x_real, x_imag, pa, pb, w0, w1, w0 - w1, bias.reshape(1, f))


def kernel(x_real, x_imag, edge_index, weight, bias):
    num_nodes = x_real.shape[0]
    src, dst = edge_index[0], edge_index[1]
    row, col, wA, wB = _coalesce_weights(edge_index, num_nodes)

    half = jnp.where(src != dst, 0.5, 0.0).astype(jnp.float32)
    deg = jnp.zeros((num_nodes,), jnp.float32).at[src].add(half).at[dst].add(half)
    dinv = jnp.where(deg > 0, lax.rsqrt(jnp.maximum(deg, 1e-30)), 0.0)

    m = row.shape[0]
    per_tile = _BLK * -(-m // (_BLK * _NTILES))
    pm = per_tile * _NTILES
    pad = pm - m
    spread = (jnp.arange(pad, dtype=jnp.int32) * 61) % num_nodes
    rowp = jnp.concatenate([row, spread])
    colp = jnp.concatenate([col, spread])
    zpad = jnp.zeros((pad,), jnp.float32)
    row_m = rowp.reshape(-1, _CHUNK)
    col_m = colp.reshape(-1, _CHUNK)
    w_m = jnp.stack([jnp.concatenate([wA, zpad]),
                     jnp.concatenate([wB, zpad])]).reshape(2, -1, _CHUNK)
    xs2 = jnp.stack([x_real, x_imag])
    out2 = _propagate_sc(xs2, dinv, row_m, col_m, w_m, num_nodes)
    npad = _NTILES * 640
    pa, pb = out2[:num_nodes], out2[npad:npad + num_nodes]
    out_real, out_imag = _dense_stage(x_real, x_imag, pa, pb,
                                      weight[0], weight[1], bias)
    return (out_real, out_imag)


# speedup vs baseline: 1.0008x; 1.0008x over previous
import functools

import jax
import jax.numpy as jnp
from jax import lax
from jax.experimental import pallas as pl
from jax.experimental.pallas import tpu as pltpu
from jax.experimental.pallas import tpu_sc as plsc

_CHUNK = 64
_CPB = 32
_BLK = _CHUNK * _CPB
_NTILES = 16
_F = 128


def _coalesce_weights(edge_index, num_nodes):
    src, dst = edge_index[0], edge_index[1]
    valid = src != dst
    kf = (src * num_nodes + dst) * 4 + jnp.where(valid, 2, 0).astype(jnp.int32)
    kr = (dst * num_nodes + src) * 4 + jnp.where(valid, 1, 0).astype(jnp.int32)
    sp = jnp.sort(jnp.concatenate([kf, kr]))
    m = sp.shape[0]
    key = sp >> 2
    tag = sp & 3
    sym_inc = (tag + 1) >> 1
    theta_inc = (tag >> 1) - (tag & 1)

    iota = jnp.arange(m, dtype=jnp.int32)
    is_last = jnp.concatenate([key[1:] != key[:-1], jnp.ones((1,), bool)])
    cs = jnp.cumsum(sym_inc, dtype=jnp.int32)
    ctp = jnp.cumsum(theta_inc + 1, dtype=jnp.int32)

    neg1 = jnp.full((1,), -1, jnp.int32)
    prev_cs = lax.cummax(jnp.concatenate([neg1, jnp.where(is_last, cs, -1)[:-1]]))
    prev_ct = lax.cummax(jnp.concatenate([neg1, jnp.where(is_last, ctp, -1)[:-1]]))
    prev_ix = lax.cummax(jnp.concatenate([neg1, jnp.where(is_last, iota, -1)[:-1]]))

    sym = cs - jnp.maximum(prev_cs, 0)
    theta = (ctp - jnp.maximum(prev_ct, 0)) - (iota - prev_ix)

    row = key // num_nodes
    col = key - row * num_nodes
    base = sym.astype(jnp.float32) * -0.5
    t4 = theta & 3
    cosv = jnp.where(t4 == 0, 1.0, jnp.where(t4 == 2, -1.0, 0.0))
    sinv = jnp.where(t4 == 1, 1.0, jnp.where(t4 == 3, -1.0, 0.0))
    wA = jnp.where(is_last, base * cosv, 0.0)
    wB = jnp.where(is_last, base * sinv, 0.0)
    return row, col, wA, wB


def _propagate_sc(xs2, dinv2, row_m, col_m, w_m, num_nodes):
    nrows = row_m.shape[0]
    nblk = nrows // (_CPB * _NTILES)
    npad = _NTILES * 640
    stripe = npad // _NTILES
    piece = _CHUNK
    mesh = plsc.VectorSubcoreMesh(core_axis_name="c", subcore_axis_name="s")

    @functools.partial(
        pl.kernel,
        mesh=mesh,
        compiler_params=pltpu.CompilerParams(needs_layout_passes=False),
        out_type=jax.ShapeDtypeStruct((2 * npad, _F), jnp.float32),
        scratch_types=[
            pltpu.VMEM_SHARED((npad, _F), jnp.float32),
            pltpu.VMEM((npad,), jnp.float32),
            pltpu.VMEM((_CPB, _CHUNK), jnp.int32),
            pltpu.VMEM((_CPB, _CHUNK), jnp.int32),
            pltpu.VMEM((_CPB, _CHUNK), jnp.float32),
            pltpu.VMEM((_CHUNK, _F), jnp.float32),
            pltpu.VMEM((_CHUNK, _F), jnp.float32),
            pltpu.SemaphoreType.DMA,
            pltpu.SemaphoreType.DMA,
            pltpu.SemaphoreType.DMA,
            pltpu.SemaphoreType.DMA,
        ],
    )
    def prop(xs_hbm, dinv_hbm, row_hbm, col_hbm, w_hbm, out_hbm,
             acc_v, dinv_v, rowb_v, colb_v, wb_v, buf_a, buf_b,
             sga, sgb, ssa, ssb):
        c = lax.axis_index("c")
        s = lax.axis_index("s")
        pltpu.sync_copy(dinv_hbm, dinv_v)
        xsc_hbm = xs_hbm.at[c]

        def zrow(i, carry):
            for k in range(_F // 16):
                buf_a[i, pl.ds(k * 16, 16)] = jnp.zeros((16,), jnp.float32)
            return carry
        lax.fori_loop(0, piece, zrow, 0)
        for p in range(stripe // piece):
            pltpu.sync_copy(buf_a, acc_v.at[pl.ds(s * stripe + p * piece, piece)])
        plsc.subcore_barrier()

        def start_gather(j, buf, sem):
            pltpu.async_copy(xsc_hbm.at[rowb_v.at[j]], buf, sem)

        def wait_gather(j, buf, sem):
            pltpu.make_async_copy(xsc_hbm.at[rowb_v.at[j]], buf, sem).wait()

        def start_scatter(j, buf, sem):
            pltpu.async_copy(buf, acc_v.at[colb_v.at[j]], sem, add=True)

        def wait_scatter(j, buf, sem):
            pltpu.make_async_copy(buf, acc_v.at[colb_v.at[j]], sem).wait()

        def scale(buf, j):
            for g in range(_CHUNK // 16):
                w16 = wb_v[j, pl.ds(g * 16, 16)]
                for lane in range(16):
                    i = g * 16 + lane
                    wsc = w16[lane]
                    for k in range(_F // 16):
                        buf[i, pl.ds(k * 16, 16)] = buf[i, pl.ds(k * 16, 16)] * wsc

        def block_body(b, carry):
            rb = (s * nblk + b) * _CPB
            pltpu.sync_copy(row_hbm.at[pl.ds(rb, _CPB)], rowb_v)
            pltpu.sync_copy(col_hbm.at[pl.ds(rb, _CPB)], colb_v)
            pltpu.sync_copy(w_hbm.at[c, pl.ds(rb, _CPB)], wb_v)

            start_gather(0, buf_a, sga)
            wait_gather(0, buf_a, sga)
            scale(buf_a, 0)
            start_scatter(0, buf_a, ssa)
            start_gather(1, buf_b, sgb)

            def steady(j2, carry2):
                j = 2 * j2 + 1
                wait_gather(j, buf_b, sgb)
                wait_scatter(j, buf_a, ssa)
                start_gather(j + 1, buf_a, sga)
                scale(buf_b, j)
                start_scatter(j, buf_b, ssb)
                jj = j + 1
                wait_gather(jj, buf_a, sga)
                wait_scatter(jj, buf_b, ssb)
                start_gather(jj + 1, buf_b, sgb)
                scale(buf_a, jj)
                start_scatter(jj, buf_a, ssa)
                return carry2
            lax.fori_loop(0, (_CPB - 2) // 2, steady, 0)

            jl = _CPB - 1
            wait_gather(jl, buf_b, sgb)
            wait_scatter(jl, buf_a, ssa)
            scale(buf_b, jl)
            start_scatter(jl, buf_b, ssb)
            wait_scatter(jl, buf_b, ssb)
            return carry
        lax.fori_loop(0, nblk, block_body, 0)
        plsc.subcore_barrier()
        for p in range(stripe // piece):
            r0 = s * stripe + p * piece
            pltpu.sync_copy(acc_v.at[pl.ds(r0, piece)], buf_a)
            for g in range(piece // 16):
                d16 = dinv_v[pl.ds(r0 + g * 16, 16)]
                for lane in range(16):
                    i = g * 16 + lane
                    dsc = d16[lane]
                    for k in range(_F // 16):
                        buf_a[i, pl.ds(k * 16, 16)] = buf_a[i, pl.ds(k * 16, 16)] * dsc
            pltpu.sync_copy(buf_a, out_hbm.at[pl.ds(c * npad + r0, piece)])

    return prop(xs2, dinv2, row_m, col_m, w_m)


def _dense_body(xr_ref, xi_ref, pa_ref, pb_ref, w0_ref, w1_ref, w01_ref,
                bias_ref, or_ref, oi_ref):
    a = jnp.dot(xr_ref[...], w0_ref[...], preferred_element_type=jnp.float32)
    a += jnp.dot(pa_ref[...], w1_ref[...], preferred_element_type=jnp.float32)
    b = jnp.dot(xi_ref[...], w01_ref[...], preferred_element_type=jnp.float32)
    b += jnp.dot(pb_ref[...], w1_ref[...], preferred_element_type=jnp.float32)
    bias = bias_ref[...]
    or_ref[...] = a - b + bias
    oi_ref[...] = a + b + bias


def _dense_stage(x_real, x_imag, pa, pb, w0, w1, bias):
    n, f = x_real.shape
    blk = 2000
    grid = n // blk
    bspec_x = pl.BlockSpec((blk, f), lambda i: (i, 0))
    bspec_w = pl.BlockSpec((f, f), lambda i: (0, 0))
    bspec_b = pl.BlockSpec((1, f), lambda i: (0, 0))
    return pl.pallas_call(
        _dense_body,
        grid=(grid,),
        in_specs=[bspec_x, bspec_x, bspec_x, bspec_x, bspec_w, bspec_w,
                  bspec_w, bspec_b],
        out_specs=[bspec_x, bspec_x],
        out_shape=[jax.ShapeDtypeStruct((n, f), jnp.float32),
                   jax.ShapeDtypeStruct((n, f), jnp.float32)],
    )(x_real, x_imag, pa, pb, w0, w1, w0 - w1, bias.reshape(1, f))


def kernel(x_real, x_imag, edge_index, weight, bias):
    num_nodes = x_real.shape[0]
    src, dst = edge_index[0], edge_index[1]
    row, col, wA, wB = _coalesce_weights(edge_index, num_nodes)

    half = jnp.where(src != dst, 0.5, 0.0).astype(jnp.float32)
    deg = jnp.zeros((num_nodes,), jnp.float32).at[src].add(half).at[dst].add(half)
    dinv = jnp.where(deg > 0, lax.rsqrt(jnp.maximum(deg, 1e-30)), 0.0)

    m = row.shape[0]
    per_tile = _BLK * -(-m // (_BLK * _NTILES))
    pm = per_tile * _NTILES
    pad = pm - m
    spread = (jnp.arange(pad, dtype=jnp.int32) * 61) % num_nodes
    rowp = jnp.concatenate([row, spread])
    colp = jnp.concatenate([col, spread])
    zpad = jnp.zeros((pad,), jnp.float32)
    row_m = rowp.reshape(-1, _CHUNK)
    col_m = colp.reshape(-1, _CHUNK)
    w_m = jnp.stack([jnp.concatenate([wA, zpad]),
                     jnp.concatenate([wB, zpad])]).reshape(2, -1, _CHUNK)
    npad = _NTILES * 640
    xs2 = jnp.stack([x_real * dinv[:, None], x_imag * dinv[:, None]])
    dinv_pad = jnp.concatenate([dinv, jnp.zeros((npad - num_nodes,), jnp.float32)])
    out2 = _propagate_sc(xs2, dinv_pad, row_m, col_m, w_m, num_nodes)
    pa, pb = out2[:num_nodes], out2[npad:npad + num_nodes]
    out_real, out_imag = _dense_stage(x_real, x_imag, pa, pb,
                                      weight[0], weight[1], bias)
    return (out_real, out_imag)


# speedup vs baseline: 1.0924x; 1.0916x over previous
import functools

import jax
import jax.numpy as jnp
from jax import lax
from jax.experimental import pallas as pl
from jax.experimental.pallas import tpu as pltpu
from jax.experimental.pallas import tpu_sc as plsc

_CHUNK = 128
_CPB = 32
_BLK = _CHUNK * _CPB
_NTILES = 16
_F = 128


def _coalesce_weights(edge_index, num_nodes):
    src, dst = edge_index[0], edge_index[1]
    valid = src != dst
    kf = (src * num_nodes + dst) * 4 + jnp.where(valid, 2, 0).astype(jnp.int32)
    kr = (dst * num_nodes + src) * 4 + jnp.where(valid, 1, 0).astype(jnp.int32)
    sp = jnp.sort(jnp.concatenate([kf, kr]))
    m = sp.shape[0]
    key = sp >> 2
    tag = sp & 3
    sym_inc = (tag + 1) >> 1
    theta_inc = (tag >> 1) - (tag & 1)

    iota = jnp.arange(m, dtype=jnp.int32)
    is_last = jnp.concatenate([key[1:] != key[:-1], jnp.ones((1,), bool)])
    cs = jnp.cumsum(sym_inc, dtype=jnp.int32)
    ctp = jnp.cumsum(theta_inc + 1, dtype=jnp.int32)

    neg1 = jnp.full((1,), -1, jnp.int32)
    prev_cs = lax.cummax(jnp.concatenate([neg1, jnp.where(is_last, cs, -1)[:-1]]))
    prev_ct = lax.cummax(jnp.concatenate([neg1, jnp.where(is_last, ctp, -1)[:-1]]))
    prev_ix = lax.cummax(jnp.concatenate([neg1, jnp.where(is_last, iota, -1)[:-1]]))

    sym = cs - jnp.maximum(prev_cs, 0)
    theta = (ctp - jnp.maximum(prev_ct, 0)) - (iota - prev_ix)

    row = key // num_nodes
    col = key - row * num_nodes
    base = sym.astype(jnp.float32) * -0.5
    t4 = theta & 3
    cosv = jnp.where(t4 == 0, 1.0, jnp.where(t4 == 2, -1.0, 0.0))
    sinv = jnp.where(t4 == 1, 1.0, jnp.where(t4 == 3, -1.0, 0.0))
    wA = jnp.where(is_last, base * cosv, 0.0)
    wB = jnp.where(is_last, base * sinv, 0.0)
    return row, col, wA, wB


def _propagate_sc(xs2, dinv2, row_m, col_m, w_m, num_nodes):
    nrows = row_m.shape[0]
    nblk = nrows // (_CPB * _NTILES)
    npad = _NTILES * 640
    stripe = npad // _NTILES
    piece = _CHUNK
    mesh = plsc.VectorSubcoreMesh(core_axis_name="c", subcore_axis_name="s")

    @functools.partial(
        pl.kernel,
        mesh=mesh,
        compiler_params=pltpu.CompilerParams(needs_layout_passes=False),
        out_type=jax.ShapeDtypeStruct((2 * npad, _F), jnp.float32),
        scratch_types=[
            pltpu.VMEM_SHARED((npad, _F), jnp.float32),
            pltpu.VMEM((npad,), jnp.float32),
            pltpu.VMEM((_CPB, _CHUNK), jnp.int32),
            pltpu.VMEM((_CPB, _CHUNK), jnp.int32),
            pltpu.VMEM((_CPB, _CHUNK), jnp.float32),
            pltpu.VMEM((_CHUNK, _F), jnp.float32),
            pltpu.SemaphoreType.DMA,
        ],
    )
    def prop(xs_hbm, dinv_hbm, row_hbm, col_hbm, w_hbm, out_hbm,
             acc_v, dinv_v, rowb_v, colb_v, wb_v, buf_a, sga):
        c = lax.axis_index("c")
        s = lax.axis_index("s")
        pltpu.sync_copy(dinv_hbm, dinv_v)
        xsc_hbm = xs_hbm.at[c]

        def zrow(i, carry):
            for k in range(_F // 16):
                buf_a[i, pl.ds(k * 16, 16)] = jnp.zeros((16,), jnp.float32)
            return carry
        lax.fori_loop(0, piece, zrow, 0)
        for p in range(stripe // piece):
            pltpu.sync_copy(buf_a, acc_v.at[pl.ds(s * stripe + p * piece, piece)])
        plsc.subcore_barrier()

        def scale(buf, j):
            for g in range(_CHUNK // 16):
                w16 = wb_v[j, pl.ds(g * 16, 16)]
                for lane in range(16):
                    i = g * 16 + lane
                    wsc = w16[lane]
                    for k in range(_F // 16):
                        buf[i, pl.ds(k * 16, 16)] = buf[i, pl.ds(k * 16, 16)] * wsc

        def block_body(b, carry):
            rb = (s * nblk + b) * _CPB
            pltpu.sync_copy(row_hbm.at[pl.ds(rb, _CPB)], rowb_v)
            pltpu.sync_copy(col_hbm.at[pl.ds(rb, _CPB)], colb_v)
            pltpu.sync_copy(w_hbm.at[c, pl.ds(rb, _CPB)], wb_v)

            def chunk_body(j, carry2):
                pltpu.async_copy(xsc_hbm.at[rowb_v.at[j]], buf_a, sga).wait()
                scale(buf_a, j)
                pltpu.sync_copy(buf_a, acc_v.at[colb_v.at[j]], add=True)
                return carry2
            lax.fori_loop(0, _CPB, chunk_body, 0)
            return carry
        lax.fori_loop(0, nblk, block_body, 0)
        plsc.subcore_barrier()

        def drain(p, carry):
            r0 = s * stripe + p * piece
            pltpu.sync_copy(acc_v.at[pl.ds(r0, piece)], buf_a)

            def dgroup(g, carry2):
                d16 = dinv_v[pl.ds(r0 + g * 16, 16)]
                for lane in range(16):
                    i = g * 16 + lane
                    dsc = d16[lane]
                    for k in range(_F // 16):
                        buf_a[i, pl.ds(k * 16, 16)] = buf_a[i, pl.ds(k * 16, 16)] * dsc
                return carry2
            lax.fori_loop(0, piece // 16, dgroup, 0)
            pltpu.sync_copy(buf_a, out_hbm.at[pl.ds(c * npad + r0, piece)])
            return carry
        lax.fori_loop(0, stripe // piece, drain, 0)

    return prop(xs2, dinv2, row_m, col_m, w_m)


def _dense_body(xr_ref, xi_ref, pa_ref, pb_ref, w0_ref, w1_ref, w01_ref,
                bias_ref, or_ref, oi_ref):
    a = jnp.dot(xr_ref[...], w0_ref[...], preferred_element_type=jnp.float32)
    a += jnp.dot(pa_ref[...], w1_ref[...], preferred_element_type=jnp.float32)
    b = jnp.dot(xi_ref[...], w01_ref[...], preferred_element_type=jnp.float32)
    b += jnp.dot(pb_ref[...], w1_ref[...], preferred_element_type=jnp.float32)
    bias = bias_ref[...]
    or_ref[...] = a - b + bias
    oi_ref[...] = a + b + bias


def _dense_stage(x_real, x_imag, pa, pb, w0, w1, bias):
    n, f = x_real.shape
    blk = 2000
    grid = n // blk
    bspec_x = pl.BlockSpec((blk, f), lambda i: (i, 0))
    bspec_w = pl.BlockSpec((f, f), lambda i: (0, 0))
    bspec_b = pl.BlockSpec((1, f), lambda i: (0, 0))
    return pl.pallas_call(
        _dense_body,
        grid=(grid,),
        in_specs=[bspec_x, bspec_x, bspec_x, bspec_x, bspec_w, bspec_w,
                  bspec_w, bspec_b],
        out_specs=[bspec_x, bspec_x],
        out_shape=[jax.ShapeDtypeStruct((n, f), jnp.float32),
                   jax.ShapeDtypeStruct((n, f), jnp.float32)],
    )(x_real, x_imag, pa, pb, w0, w1, w0 - w1, bias.reshape(1, f))


def kernel(x_real, x_imag, edge_index, weight, bias):
    num_nodes = x_real.shape[0]
    src, dst = edge_index[0], edge_index[1]
    row, col, wA, wB = _coalesce_weights(edge_index, num_nodes)

    half = jnp.where(src != dst, 0.5, 0.0).astype(jnp.float32)
    deg = jnp.zeros((num_nodes,), jnp.float32).at[src].add(half).at[dst].add(half)
    dinv = jnp.where(deg > 0, lax.rsqrt(jnp.maximum(deg, 1e-30)), 0.0)

    m = row.shape[0]
    per_tile = _BLK * -(-m // (_BLK * _NTILES))
    pm = per_tile * _NTILES
    pad = pm - m
    spread = (jnp.arange(pad, dtype=jnp.int32) * 61) % num_nodes
    rowp = jnp.concatenate([row, spread])
    colp = jnp.concatenate([col, spread])
    zpad = jnp.zeros((pad,), jnp.float32)
    row_m = rowp.reshape(-1, _CHUNK)
    col_m = colp.reshape(-1, _CHUNK)
    w_m = jnp.stack([jnp.concatenate([wA, zpad]),
                     jnp.concatenate([wB, zpad])]).reshape(2, -1, _CHUNK)
    npad = _NTILES * 640
    xs2 = jnp.stack([x_real * dinv[:, None], x_imag * dinv[:, None]])
    dinv_pad = jnp.concatenate([dinv, jnp.zeros((npad - num_nodes,), jnp.float32)])
    out2 = _propagate_sc(xs2, dinv_pad, row_m, col_m, w_m, num_nodes)
    pa, pb = out2[:num_nodes], out2[npad:npad + num_nodes]
    out_real, out_imag = _dense_stage(x_real, x_imag, pa, pb,
                                      weight[0], weight[1], bias)
    return (out_real, out_imag)
